# Initial kernel scaffold; baseline (speedup 1.0000x reference)
import jax
import jax.numpy as jnp
from jax.experimental import pallas as pl


def kernel(x, edge_index, W1, b1, W2, b2):
    raise NotImplementedError("write your pallas kernel here")


# speedup vs baseline: 14.0514x; 14.0514x over previous
import dataclasses
import functools

import jax
import jax.numpy as jnp
from jax import lax
from jax.experimental import pallas as pl
from jax.experimental.pallas import tpu as pltpu
from jax.experimental.pallas import tpu_sc as plsc

N = 10000
E = 320000
D = 128

NC = 2
NS = 16
L = 16
NW = NC * NS
EPW = E // NW
K = 80
NBLK = EPW // K
NPAD = 10240
ROWS_PT = NPAD // NS

_MESH = plsc.VectorSubcoreMesh(core_axis_name="c", subcore_axis_name="s")

_SC_PARAMS = pltpu.CompilerParams()
if "needs_layout_passes" in pltpu.CompilerParams.__dataclass_fields__:
    _SC_PARAMS = dataclasses.replace(_SC_PARAMS, needs_layout_passes=False)


@functools.partial(
    pl.kernel,
    out_type=jax.ShapeDtypeStruct((NW, N), jnp.float32),
    mesh=_MESH,
    compiler_params=_SC_PARAMS,
    scratch_types=[
        pltpu.VMEM((EPW,), jnp.int32),
        pltpu.VMEM((N,), jnp.float32),
    ],
)
def _deg_kernel(dst_hbm, out_hbm, idx_v, hist_v):
    wid = lax.axis_index("s") * NC + lax.axis_index("c")

    @pl.loop(0, N, step=L)
    def _(i):
        hist_v[pl.ds(i, L)] = jnp.zeros((L,), jnp.float32)

    pltpu.sync_copy(dst_hbm.at[pl.ds(wid * EPW, EPW)], idx_v)
    ones = jnp.ones((L,), jnp.float32)

    @pl.loop(0, EPW, step=L)
    def _(i):
        plsc.addupdate_scatter(hist_v, [idx_v[pl.ds(i, L)]], ones)

    pltpu.sync_copy(hist_v, out_hbm.at[wid])


@functools.partial(
    pl.kernel,
    out_type=jax.ShapeDtypeStruct((NC, NPAD, D), jnp.float32),
    mesh=_MESH,
    compiler_params=_SC_PARAMS,
    scratch_types=[
        pltpu.VMEM((K,), jnp.int32),
        pltpu.VMEM((K,), jnp.int32),
        pltpu.VMEM((K, D), jnp.float32),
        pltpu.VMEM((ROWS_PT // 5, D), jnp.float32),
        pltpu.VMEM_SHARED((NPAD, D), jnp.float32),
        pltpu.SemaphoreType.DMA,
    ],
)
def _edge_kernel(y_hbm, src_hbm, dst_hbm, out_hbm,
                 src_v, dst_v, rows_v, zero_v, z_sh, sem):
    cid = lax.axis_index("c")
    sid = lax.axis_index("s")
    wid = sid * NC + cid
    zchunk = ROWS_PT // 5

    @pl.loop(0, zchunk)
    def _(r):
        @pl.loop(0, D, step=L)
        def _(c):
            zero_v[r, pl.ds(c, L)] = jnp.zeros((L,), jnp.float32)

    @pl.loop(0, 5)
    def _(j):
        pltpu.sync_copy(zero_v, z_sh.at[pl.ds(sid * ROWS_PT + j * zchunk, zchunk)])

    plsc.subcore_barrier()

    @pl.loop(0, NBLK)
    def _(blk):
        pltpu.sync_copy(src_hbm.at[wid, blk], src_v)
        pltpu.sync_copy(dst_hbm.at[wid, blk], dst_v)
        pltpu.async_copy(y_hbm.at[src_v], rows_v, sem).wait()
        pltpu.sync_copy(rows_v, z_sh.at[dst_v], add=True)

    plsc.subcore_barrier()
    pltpu.sync_copy(
        z_sh.at[pl.ds(sid * ROWS_PT, ROWS_PT)],
        out_hbm.at[cid, pl.ds(sid * ROWS_PT, ROWS_PT)],
    )


def _matmul(x, w):
    def body(x_ref, w_ref, o_ref):
        o_ref[...] = lax.dot_general(
            x_ref[...], w_ref[...], (((1,), (0,)), ((), ())),
            precision=lax.Precision.HIGHEST,
            preferred_element_type=jnp.float32,
        )

    return pl.pallas_call(
        body,
        out_shape=jax.ShapeDtypeStruct((x.shape[0], w.shape[1]), jnp.float32),
    )(x, w)


def _scale_kernel(degp, t1):
    def body(degp_ref, t_ref, dinv_ref, y_ref):
        deg = jnp.sum(degp_ref[...], axis=0) + 1.0
        dinv = lax.rsqrt(deg)[:, None]
        dinv_ref[...] = dinv
        y_ref[...] = t_ref[...] * dinv

    return pl.pallas_call(
        body,
        out_shape=(
            jax.ShapeDtypeStruct((N, 1), jnp.float32),
            jax.ShapeDtypeStruct((N, D), jnp.float32),
        ),
    )(degp, t1)


def _mid_kernel(z, y1, dinv, b1, w2):
    def body(z_ref, y_ref, dinv_ref, b_ref, w_ref, y2_ref):
        ztot = z_ref[0, :N] + z_ref[1, :N] + y_ref[...]
        h = jnp.maximum(ztot * dinv_ref[...] + b_ref[...], 0.0)
        t2 = lax.dot_general(
            h, w_ref[...], (((1,), (0,)), ((), ())),
            precision=lax.Precision.HIGHEST,
            preferred_element_type=jnp.float32,
        )
        y2_ref[...] = t2 * dinv_ref[...]

    return pl.pallas_call(
        body,
        out_shape=jax.ShapeDtypeStruct((N, D), jnp.float32),
    )(z, y1, dinv, b1, w2)


def _final_kernel(z, y2, dinv, b2):
    def body(z_ref, y_ref, dinv_ref, b_ref, o_ref):
        ztot = z_ref[0, :N] + z_ref[1, :N] + y_ref[...]
        o_ref[...] = ztot * dinv_ref[...] + b_ref[...]

    return pl.pallas_call(
        body,
        out_shape=jax.ShapeDtypeStruct((N, D), jnp.float32),
    )(z, y2, dinv, b2)


def kernel(x, edge_index, W1, b1, W2, b2):
    ei = edge_index.astype(jnp.int32)
    src3 = ei[0].reshape(NW, NBLK, K)
    dst_flat = ei[1]
    dst3 = dst_flat.reshape(NW, NBLK, K)

    degp = _deg_kernel(dst_flat)
    t1 = _matmul(x, W1)
    dinv, y1 = _scale_kernel(degp, t1)
    z1 = _edge_kernel(y1, src3, dst3)
    y2 = _mid_kernel(z1, y1, dinv, b1.reshape(1, D), W2)
    z2 = _edge_kernel(y2, src3, dst3)
    return _final_kernel(z2, y2, dinv, b2.reshape(1, D))


# speedup vs baseline: 38.9946x; 2.7751x over previous
import dataclasses
import functools

import jax
import jax.numpy as jnp
from jax import lax
from jax.experimental import pallas as pl
from jax.experimental.pallas import tpu as pltpu
from jax.experimental.pallas import tpu_sc as plsc

N = 10000
E = 320000
D = 128

NC = 2
NS = 16
L = 16
NW = NC * NS
EPW = E // NW
K = 40
NBLK = EPW // K
NPAD = 10240
ROWS_PT = NPAD // NS

_MESH = plsc.VectorSubcoreMesh(core_axis_name="c", subcore_axis_name="s")

_SC_PARAMS = pltpu.CompilerParams()
if "needs_layout_passes" in pltpu.CompilerParams.__dataclass_fields__:
    _SC_PARAMS = dataclasses.replace(_SC_PARAMS, needs_layout_passes=False)


@functools.partial(
    pl.kernel,
    out_type=jax.ShapeDtypeStruct((NW, N), jnp.float32),
    mesh=_MESH,
    compiler_params=_SC_PARAMS,
    scratch_types=[
        pltpu.VMEM((EPW,), jnp.int32),
        pltpu.VMEM((N,), jnp.float32),
    ],
)
def _deg_kernel(dst_hbm, out_hbm, idx_v, hist_v):
    wid = lax.axis_index("s") * NC + lax.axis_index("c")

    @pl.loop(0, N, step=L)
    def _(i):
        hist_v[pl.ds(i, L)] = jnp.zeros((L,), jnp.float32)

    pltpu.sync_copy(dst_hbm.at[pl.ds(wid * EPW, EPW)], idx_v)
    ones = jnp.ones((L,), jnp.float32)

    @pl.loop(0, EPW, step=L)
    def _(i):
        plsc.addupdate_scatter(hist_v, [idx_v[pl.ds(i, L)]], ones)

    pltpu.sync_copy(hist_v, out_hbm.at[wid])


NBUF = 5
ZCH = K


@functools.partial(
    pl.kernel,
    out_type=jax.ShapeDtypeStruct((NC, NPAD, D), jnp.float32),
    mesh=_MESH,
    compiler_params=_SC_PARAMS,
    scratch_types=[
        pltpu.VMEM((EPW,), jnp.int32),
        pltpu.VMEM((EPW,), jnp.int32),
        pltpu.VMEM((NBUF, K, D), jnp.float32),
        pltpu.VMEM_SHARED((NPAD, D), jnp.float32),
        pltpu.SemaphoreType.DMA,
        pltpu.SemaphoreType.DMA,
        pltpu.SemaphoreType.DMA,
        pltpu.SemaphoreType.DMA,
        pltpu.SemaphoreType.DMA,
    ],
)
def _edge_kernel(y_hbm, src_hbm, dst_hbm, out_hbm,
                 src_v, dst_v, rows_v, z_sh, *sems):
    cid = lax.axis_index("c")
    sid = lax.axis_index("s")
    wid = sid * NC + cid

    pltpu.sync_copy(src_hbm.at[pl.ds(wid * EPW, EPW)], src_v)
    pltpu.sync_copy(dst_hbm.at[pl.ds(wid * EPW, EPW)], dst_v)

    @pl.loop(0, ZCH)
    def _(r):
        @pl.loop(0, D, step=L)
        def _(c):
            rows_v[0, r, pl.ds(c, L)] = jnp.zeros((L,), jnp.float32)

    @pl.loop(0, ROWS_PT // ZCH)
    def _(j):
        pltpu.sync_copy(rows_v.at[0, pl.ds(0, ZCH)],
                        z_sh.at[pl.ds(sid * ROWS_PT + j * ZCH, ZCH)])

    plsc.subcore_barrier()

    def gather(blk, b):
        return pltpu.make_async_copy(y_hbm.at[src_v.at[pl.ds(blk * K, K)]],
                                     rows_v.at[b], sems[b])

    def scatter(blk, b):
        pltpu.sync_copy(rows_v.at[b],
                        z_sh.at[dst_v.at[pl.ds(blk * K, K)]], add=True)

    for b in range(NBUF):
        gather(b, b).start()

    @pl.loop(0, NBLK - NBUF, step=NBUF)
    def _(t):
        for b in range(NBUF):
            gather(t + b, b).wait()
            scatter(t + b, b)
            gather(t + b + NBUF, b).start()

    t_last = NBLK - NBUF
    for b in range(NBUF):
        gather(t_last + b, b).wait()
        scatter(t_last + b, b)

    plsc.subcore_barrier()
    pltpu.sync_copy(
        z_sh.at[pl.ds(sid * ROWS_PT, ROWS_PT)],
        out_hbm.at[cid, pl.ds(sid * ROWS_PT, ROWS_PT)],
    )


def _matmul(x, w):
    def body(x_ref, w_ref, o_ref):
        o_ref[...] = lax.dot_general(
            x_ref[...], w_ref[...], (((1,), (0,)), ((), ())),
            precision=lax.Precision.HIGHEST,
            preferred_element_type=jnp.float32,
        )

    return pl.pallas_call(
        body,
        out_shape=jax.ShapeDtypeStruct((x.shape[0], w.shape[1]), jnp.float32),
    )(x, w)


def _scale_kernel(degp, t1):
    def body(degp_ref, t_ref, dinv_ref, y_ref):
        deg = jnp.sum(degp_ref[...], axis=0) + 1.0
        dinv = lax.rsqrt(deg)[:, None]
        dinv_ref[...] = dinv
        y_ref[...] = t_ref[...] * dinv

    return pl.pallas_call(
        body,
        out_shape=(
            jax.ShapeDtypeStruct((N, 1), jnp.float32),
            jax.ShapeDtypeStruct((N, D), jnp.float32),
        ),
    )(degp, t1)


def _mid_kernel(z, y1, dinv, b1, w2):
    def body(z_ref, y_ref, dinv_ref, b_ref, w_ref, y2_ref):
        ztot = z_ref[0, :N] + z_ref[1, :N] + y_ref[...]
        h = jnp.maximum(ztot * dinv_ref[...] + b_ref[...], 0.0)
        t2 = lax.dot_general(
            h, w_ref[...], (((1,), (0,)), ((), ())),
            precision=lax.Precision.HIGHEST,
            preferred_element_type=jnp.float32,
        )
        y2_ref[...] = t2 * dinv_ref[...]

    return pl.pallas_call(
        body,
        out_shape=jax.ShapeDtypeStruct((N, D), jnp.float32),
    )(z, y1, dinv, b1, w2)


def _final_kernel(z, y2, dinv, b2):
    def body(z_ref, y_ref, dinv_ref, b_ref, o_ref):
        ztot = z_ref[0, :N] + z_ref[1, :N] + y_ref[...]
        o_ref[...] = ztot * dinv_ref[...] + b_ref[...]

    return pl.pallas_call(
        body,
        out_shape=jax.ShapeDtypeStruct((N, D), jnp.float32),
    )(z, y2, dinv, b2)


def kernel(x, edge_index, W1, b1, W2, b2):
    ei = edge_index.astype(jnp.int32)
    src = ei[0]
    dst = ei[1]

    degp = _deg_kernel(dst)
    t1 = _matmul(x, W1)
    dinv, y1 = _scale_kernel(degp, t1)
    z1 = _edge_kernel(y1, src, dst)
    y2 = _mid_kernel(z1, y1, dinv, b1.reshape(1, D), W2)
    z2 = _edge_kernel(y2, src, dst)
    return _final_kernel(z2, y2, dinv, b2.reshape(1, D))
